# baseline (speedup 1.0000x reference)
import functools

import jax
import jax.numpy as jnp
from jax import lax
from jax.experimental import pallas as pl
from jax.experimental.pallas import tpu as pltpu
from jax.experimental.pallas import tpu_sc as plsc

N = 100000
D = 128
L = 16
NC = 2
NS = 16
NW = NC * NS
BR = 32
NCH = N // BR
CH_BASE = NCH // NW
CH_REM = NCH % NW
NCC = CH_BASE + 1
NR = 4
SLOTS = NR * ((NCC + NR - 1) // NR)
CH = D // L

_mesh = plsc.VectorSubcoreMesh(core_axis_name="c", subcore_axis_name="s")


def _merge16(tk, tv, bk, bv):
    rk = lax.rev(bk, (0,))
    rv = lax.rev(bv, (0,))
    m = rk < tk
    nk = jnp.where(m, rk, tk)
    nv = jnp.where(m, rv, tv)
    sk, sv = plsc.sort_key_val(nk, nv)
    return sk, sv


@functools.partial(
    pl.kernel,
    mesh=_mesh,
    compiler_params=pltpu.CompilerParams(needs_layout_passes=False),
    out_type=[
        jax.ShapeDtypeStruct((NW, L), jnp.float32),
        jax.ShapeDtypeStruct((NW, L), jnp.int32),
    ],
    scratch_types=[
        pltpu.VMEM((D,), jnp.float32),
        pltpu.VMEM((BR, D), jnp.float32),
        pltpu.VMEM((BR, D), jnp.float32),
        pltpu.VMEM((BR, D), jnp.float32),
        pltpu.VMEM((BR, D), jnp.float32),
        pltpu.VMEM((L,), jnp.float32),
        pltpu.VMEM((L,), jnp.int32),
        pltpu.SemaphoreType.DMA,
        pltpu.SemaphoreType.DMA,
        pltpu.SemaphoreType.DMA,
        pltpu.SemaphoreType.DMA,
    ],
)
def _local_topk(x_hbm, data_hbm, okeys_hbm, ovals_hbm,
                x_v, buf0_v, buf1_v, buf2_v, buf3_v, k_v, v_v,
                sem0, sem1, sem2, sem3):
    wid = lax.axis_index("s") * NC + lax.axis_index("c")
    nch = CH_BASE + jnp.where(wid >= NW - CH_REM, 1, 0)
    start = CH_BASE * wid + jnp.maximum(wid - (NW - CH_REM), 0)

    pltpu.sync_copy(x_hbm, x_v)
    xs = [x_v[pl.ds(c * L, L)] for c in range(CH)]
    iota = lax.broadcasted_iota(jnp.int32, (L,), 0)
    bufs = (buf0_v, buf1_v, buf2_v, buf3_v)
    sems = (sem0, sem1, sem2, sem3)

    def issue(g, b):
        @pl.when(g < NCC)
        def _():
            pltpu.async_copy(
                data_hbm.at[pl.ds((start + g) * BR, BR), :], bufs[b], sems[b])

    for b in range(NR):
        issue(jnp.int32(b), b)

    tk0 = jnp.full((L,), jnp.inf, jnp.float32)
    tv0 = jnp.zeros((L,), jnp.int32)

    def group_sums(buf_v, half):
        sums = []
        for j in range(L):
            sq = []
            for c in range(CH):
                dc = buf_v[half * L + j, pl.ds(c * L, L)] - xs[c]
                sq.append(dc * dc)
            acc = ((sq[0] + sq[1]) + (sq[2] + sq[3])) + (
                (sq[4] + sq[5]) + (sq[6] + sq[7]))
            sums.append(jnp.sum(acc))
        return sums

    def select_tree(sums):
        sel = [jnp.where(iota == (2 * j), sums[2 * j], sums[2 * j + 1])
               for j in range(8)]
        sel = [jnp.where((iota >> 1) == (2 * j), sel[2 * j], sel[2 * j + 1])
               for j in range(4)]
        sel = [jnp.where((iota >> 2) == (2 * j), sel[2 * j], sel[2 * j + 1])
               for j in range(2)]
        return jnp.where(iota < 8, sel[0], sel[1])

    def body(gg, carry):
        tk, tv = carry
        for b in range(NR):
            g = gg * NR + b
            buf_v = bufs[b]

            @pl.when(g < NCC)
            def _():
                pltpu.make_async_copy(
                    data_hbm.at[pl.ds(0, BR), :], buf_v, sems[b]).wait()

            row0 = (start + g) * BR
            sums0 = group_sums(buf_v, 0)
            sums1 = group_sums(buf_v, 1)
            issue(g + NR, b)
            mn = functools.reduce(jnp.minimum, sums0 + sums1)
            thresh = jnp.max(tk)
            do = jnp.logical_and(mn < thresh, g < nch)

            def do_merge():
                d0 = select_tree(sums0)
                d1 = select_tree(sums1)
                bk0, bv0 = plsc.sort_key_val(d0, row0 + iota)
                bk1, bv1 = plsc.sort_key_val(d1, row0 + L + iota)
                bk, bv = _merge16(bk0, bv0, bk1, bv1)
                return _merge16(tk, tv, bk, bv)

            tk, tv = lax.cond(do, do_merge, lambda: (tk, tv))
        return tk, tv

    tk, tv = lax.fori_loop(0, SLOTS // NR, body, (tk0, tv0))
    k_v[...] = tk
    v_v[...] = tv
    pltpu.sync_copy(k_v, okeys_hbm.at[wid])
    pltpu.sync_copy(v_v, ovals_hbm.at[wid])


@functools.partial(
    pl.kernel,
    mesh=_mesh,
    compiler_params=pltpu.CompilerParams(needs_layout_passes=False),
    out_type=jax.ShapeDtypeStruct((L,), jnp.int32),
    scratch_types=[
        pltpu.VMEM((NW, L), jnp.float32),
        pltpu.VMEM((NW, L), jnp.int32),
        pltpu.VMEM((L,), jnp.int32),
        pltpu.VMEM((L,), jnp.int32),
        pltpu.SemaphoreType.DMA,
    ],
)
def _final_mode(keys_hbm, vals_hbm, tgt_hbm, out_hbm, k_v, v_v, t_v, o_v, sem):
    wid = lax.axis_index("s") * NC + lax.axis_index("c")
    iota = lax.broadcasted_iota(jnp.int32, (L,), 0)

    @pl.when(wid == 0)
    def _():
        pltpu.sync_copy(keys_hbm, k_v)
        pltpu.sync_copy(vals_hbm, v_v)
        tk = k_v[0, :]
        tv = v_v[0, :]
        for w in range(1, NW):
            tk, tv = _merge16(tk, tv, k_v[w, :], v_v[w, :])
        pltpu.async_copy(tgt_hbm.at[tv], t_v, sem).wait()
        tg = t_v[...]
        cnt = jnp.zeros((L,), jnp.int32)
        for i in range(L):
            ti = jnp.max(jnp.where(iota == i, tg, 0))
            cnt = cnt + jnp.where(tg == ti, 1, 0)
        score = cnt * 2048 + (1023 - tg)
        mx = jnp.max(score)
        pred = 1023 - (mx & 2047)
        o_v[...] = jnp.full((L,), pred, jnp.int32)
        pltpu.sync_copy(o_v, out_hbm)


def kernel(x, train_data, train_targets):
    tt = train_targets.astype(jnp.int32)
    keys, vals = _local_topk(x, train_data)
    out = _final_mode(keys, vals, tt)
    return out[0]


# speedup vs baseline: 1.0437x; 1.0437x over previous
import functools

import jax
import jax.numpy as jnp
from jax import lax
from jax.experimental import pallas as pl
from jax.experimental.pallas import tpu as pltpu
from jax.experimental.pallas import tpu_sc as plsc

N = 100000
D = 128
L = 16
NC = 2
NS = 16
NW = NC * NS
BR = 32
NCH = N // BR
CH_BASE = NCH // NW
CH_REM = NCH % NW
NCC = CH_BASE + 1
NR = 2
SLOTS = NCC
CH = D // L

_mesh = plsc.VectorSubcoreMesh(core_axis_name="c", subcore_axis_name="s")


def _merge16(tk, tv, bk, bv):
    rk = lax.rev(bk, (0,))
    rv = lax.rev(bv, (0,))
    m = rk < tk
    nk = jnp.where(m, rk, tk)
    nv = jnp.where(m, rv, tv)
    sk, sv = plsc.sort_key_val(nk, nv)
    return sk, sv


@functools.partial(
    pl.kernel,
    mesh=_mesh,
    compiler_params=pltpu.CompilerParams(needs_layout_passes=False),
    out_type=[
        jax.ShapeDtypeStruct((NW, L), jnp.float32),
        jax.ShapeDtypeStruct((NW, L), jnp.int32),
    ],
    scratch_types=[
        pltpu.VMEM((D,), jnp.float32),
        pltpu.VMEM((BR, D), jnp.float32),
        pltpu.VMEM((BR, D), jnp.float32),
        pltpu.VMEM((L,), jnp.float32),
        pltpu.VMEM((L,), jnp.int32),
        pltpu.SemaphoreType.DMA,
        pltpu.SemaphoreType.DMA,
    ],
)
def _local_topk(x_hbm, data_hbm, okeys_hbm, ovals_hbm,
                x_v, buf0_v, buf1_v, k_v, v_v, sem0, sem1):
    wid = lax.axis_index("s") * NC + lax.axis_index("c")
    nch = CH_BASE + jnp.where(wid >= NW - CH_REM, 1, 0)
    start = CH_BASE * wid + jnp.maximum(wid - (NW - CH_REM), 0)

    pltpu.sync_copy(x_hbm, x_v)
    xs = [x_v[pl.ds(c * L, L)] for c in range(CH)]
    iota = lax.broadcasted_iota(jnp.int32, (L,), 0)
    bufs = (buf0_v, buf1_v)
    sems = (sem0, sem1)

    def issue(g, b):
        ci = jnp.minimum(start + g, NCH - 1)
        pltpu.async_copy(
            data_hbm.at[pl.ds(ci * BR, BR), :], bufs[b], sems[b])

    for b in range(NR):
        issue(jnp.int32(b), b)

    tk0 = jnp.full((L,), jnp.inf, jnp.float32)
    tv0 = jnp.zeros((L,), jnp.int32)

    def group_dist(buf_v, half):
        sums = []
        for j in range(L):
            sq = []
            for c in range(CH):
                dc = buf_v[half * L + j, pl.ds(c * L, L)] - xs[c]
                sq.append(dc * dc)
            acc = ((sq[0] + sq[1]) + (sq[2] + sq[3])) + (
                (sq[4] + sq[5]) + (sq[6] + sq[7]))
            sums.append(jnp.sum(acc))
        sel = [jnp.where(iota == (2 * j), sums[2 * j], sums[2 * j + 1])
               for j in range(8)]
        sel = [jnp.where((iota >> 1) == (2 * j), sel[2 * j], sel[2 * j + 1])
               for j in range(4)]
        sel = [jnp.where((iota >> 2) == (2 * j), sel[2 * j], sel[2 * j + 1])
               for j in range(2)]
        return jnp.where(iota < 8, sel[0], sel[1])

    def body(gg, carry):
        tk, tv = carry
        for b in range(NR):
            g = gg * NR + b
            buf_v = bufs[b]
            pltpu.make_async_copy(
                data_hbm.at[pl.ds(0, BR), :], buf_v, sems[b]).wait()
            row0 = (start + g) * BR
            d0 = group_dist(buf_v, 0)
            d1 = group_dist(buf_v, 1)
            issue(g + NR, b)
            valid = g < nch
            d0 = jnp.where(valid, d0, jnp.inf)
            d1 = jnp.where(valid, d1, jnp.inf)
            bk0, bv0 = plsc.sort_key_val(d0, row0 + iota)
            bk1, bv1 = plsc.sort_key_val(d1, row0 + L + iota)
            bk, bv = _merge16(bk0, bv0, bk1, bv1)
            tk, tv = _merge16(tk, tv, bk, bv)
        return tk, tv

    tk, tv = lax.fori_loop(0, SLOTS // NR, body, (tk0, tv0))
    for b in range(NR):
        pltpu.make_async_copy(
            data_hbm.at[pl.ds(0, BR), :], bufs[b], sems[b]).wait()
    k_v[...] = tk
    v_v[...] = tv
    pltpu.sync_copy(k_v, okeys_hbm.at[wid])
    pltpu.sync_copy(v_v, ovals_hbm.at[wid])


@functools.partial(
    pl.kernel,
    mesh=_mesh,
    compiler_params=pltpu.CompilerParams(needs_layout_passes=False),
    out_type=jax.ShapeDtypeStruct((L,), jnp.int32),
    scratch_types=[
        pltpu.VMEM((NW, L), jnp.float32),
        pltpu.VMEM((NW, L), jnp.int32),
        pltpu.VMEM((L,), jnp.int32),
        pltpu.VMEM((L,), jnp.int32),
        pltpu.SemaphoreType.DMA,
    ],
)
def _final_mode(keys_hbm, vals_hbm, tgt_hbm, out_hbm, k_v, v_v, t_v, o_v, sem):
    wid = lax.axis_index("s") * NC + lax.axis_index("c")
    iota = lax.broadcasted_iota(jnp.int32, (L,), 0)

    @pl.when(wid == 0)
    def _():
        pltpu.sync_copy(keys_hbm, k_v)
        pltpu.sync_copy(vals_hbm, v_v)
        tk = k_v[0, :]
        tv = v_v[0, :]
        for w in range(1, NW):
            tk, tv = _merge16(tk, tv, k_v[w, :], v_v[w, :])
        pltpu.async_copy(tgt_hbm.at[tv], t_v, sem).wait()
        tg = t_v[...]
        cnt = jnp.zeros((L,), jnp.int32)
        for i in range(L):
            ti = jnp.max(jnp.where(iota == i, tg, 0))
            cnt = cnt + jnp.where(tg == ti, 1, 0)
        score = cnt * 2048 + (1023 - tg)
        mx = jnp.max(score)
        pred = 1023 - (mx & 2047)
        o_v[...] = jnp.full((L,), pred, jnp.int32)
        pltpu.sync_copy(o_v, out_hbm)


def kernel(x, train_data, train_targets):
    tt = train_targets.astype(jnp.int32)
    keys, vals = _local_topk(x, train_data)
    out = _final_mode(keys, vals, tt)
    return out[0]


# speedup vs baseline: 1.3966x; 1.3381x over previous
import functools

import jax
import jax.numpy as jnp
from jax import lax
from jax.experimental import pallas as pl
from jax.experimental.pallas import tpu as pltpu
from jax.experimental.pallas import tpu_sc as plsc

N = 100000
D = 128
L = 16
NC = 2
NS = 16
NW = NC * NS
BR = 32
NCH = N // BR
CH_BASE = NCH // NW
CH_REM = NCH % NW
NCC = CH_BASE + 1
NR = 4
SLOTS = NR * ((NCC + NR - 1) // NR)
CH = D // L

_mesh = plsc.VectorSubcoreMesh(core_axis_name="c", subcore_axis_name="s")


def _merge16(tk, tv, bk, bv):
    rk = lax.rev(bk, (0,))
    rv = lax.rev(bv, (0,))
    m = rk < tk
    nk = jnp.where(m, rk, tk)
    nv = jnp.where(m, rv, tv)
    sk, sv = plsc.sort_key_val(nk, nv)
    return sk, sv


@functools.partial(
    pl.kernel,
    mesh=_mesh,
    compiler_params=pltpu.CompilerParams(needs_layout_passes=False),
    out_type=[
        jax.ShapeDtypeStruct((NW, L), jnp.float32),
        jax.ShapeDtypeStruct((NW, L), jnp.int32),
    ],
    scratch_types=[
        pltpu.VMEM((D,), jnp.float32),
        pltpu.VMEM((BR, D), jnp.float32),
        pltpu.VMEM((BR, D), jnp.float32),
        pltpu.VMEM((BR, D), jnp.float32),
        pltpu.VMEM((BR, D), jnp.float32),
        pltpu.VMEM((L,), jnp.float32),
        pltpu.VMEM((L,), jnp.int32),
        pltpu.SemaphoreType.DMA,
        pltpu.SemaphoreType.DMA,
        pltpu.SemaphoreType.DMA,
        pltpu.SemaphoreType.DMA,
    ],
)
def _local_topk(x_hbm, data_hbm, okeys_hbm, ovals_hbm,
                x_v, buf0_v, buf1_v, buf2_v, buf3_v, k_v, v_v,
                sem0, sem1, sem2, sem3):
    wid = lax.axis_index("s") * NC + lax.axis_index("c")
    nch = CH_BASE + jnp.where(wid >= NW - CH_REM, 1, 0)
    start = CH_BASE * wid + jnp.maximum(wid - (NW - CH_REM), 0)

    pltpu.sync_copy(x_hbm, x_v)
    xs = [x_v[pl.ds(c * L, L)] for c in range(CH)]
    iota = lax.broadcasted_iota(jnp.int32, (L,), 0)
    bufs = (buf0_v, buf1_v, buf2_v, buf3_v)
    sems = (sem0, sem1, sem2, sem3)

    def issue(g, b):
        ci = jnp.minimum(start + g, NCH - 1)
        pltpu.async_copy(
            data_hbm.at[pl.ds(ci * BR, BR), :], bufs[b], sems[b])

    for b in range(NR):
        issue(jnp.int32(b), b)

    tk0 = jnp.full((L,), jnp.inf, jnp.float32)
    tv0 = jnp.zeros((L,), jnp.int32)

    def group_dist(buf_v, half):
        sums = []
        for j in range(L):
            sq = []
            for c in range(CH):
                dc = buf_v[half * L + j, pl.ds(c * L, L)] - xs[c]
                sq.append(dc * dc)
            acc = ((sq[0] + sq[1]) + (sq[2] + sq[3])) + (
                (sq[4] + sq[5]) + (sq[6] + sq[7]))
            sums.append(jnp.sum(acc))
        sel = [jnp.where(iota == (2 * j), sums[2 * j], sums[2 * j + 1])
               for j in range(8)]
        sel = [jnp.where((iota >> 1) == (2 * j), sel[2 * j], sel[2 * j + 1])
               for j in range(4)]
        sel = [jnp.where((iota >> 2) == (2 * j), sel[2 * j], sel[2 * j + 1])
               for j in range(2)]
        return jnp.where(iota < 8, sel[0], sel[1])

    def body(gg, carry):
        tk, tv = carry
        for b in range(NR):
            g = gg * NR + b
            buf_v = bufs[b]
            pltpu.make_async_copy(
                data_hbm.at[pl.ds(0, BR), :], buf_v, sems[b]).wait()
            row0 = (start + g) * BR
            d0 = group_dist(buf_v, 0)
            d1 = group_dist(buf_v, 1)
            issue(g + NR, b)
            valid = g < nch
            d0 = jnp.where(valid, d0, jnp.inf)
            d1 = jnp.where(valid, d1, jnp.inf)
            bk0, bv0 = plsc.sort_key_val(d0, row0 + iota)
            bk1, bv1 = plsc.sort_key_val(d1, row0 + L + iota)
            bk, bv = _merge16(bk0, bv0, bk1, bv1)
            tk, tv = _merge16(tk, tv, bk, bv)
        return tk, tv

    tk, tv = lax.fori_loop(0, SLOTS // NR, body, (tk0, tv0))
    for b in range(NR):
        pltpu.make_async_copy(
            data_hbm.at[pl.ds(0, BR), :], bufs[b], sems[b]).wait()
    k_v[...] = tk
    v_v[...] = tv
    pltpu.sync_copy(k_v, okeys_hbm.at[wid])
    pltpu.sync_copy(v_v, ovals_hbm.at[wid])


@functools.partial(
    pl.kernel,
    mesh=_mesh,
    compiler_params=pltpu.CompilerParams(needs_layout_passes=False),
    out_type=jax.ShapeDtypeStruct((L,), jnp.int32),
    scratch_types=[
        pltpu.VMEM((NW, L), jnp.float32),
        pltpu.VMEM((NW, L), jnp.int32),
        pltpu.VMEM((L,), jnp.int32),
        pltpu.VMEM((L,), jnp.int32),
        pltpu.SemaphoreType.DMA,
    ],
)
def _final_mode(keys_hbm, vals_hbm, tgt_hbm, out_hbm, k_v, v_v, t_v, o_v, sem):
    wid = lax.axis_index("s") * NC + lax.axis_index("c")
    iota = lax.broadcasted_iota(jnp.int32, (L,), 0)

    @pl.when(wid == 0)
    def _():
        pltpu.sync_copy(keys_hbm, k_v)
        pltpu.sync_copy(vals_hbm, v_v)
        tk = k_v[0, :]
        tv = v_v[0, :]
        for w in range(1, NW):
            tk, tv = _merge16(tk, tv, k_v[w, :], v_v[w, :])
        pltpu.async_copy(tgt_hbm.at[tv], t_v, sem).wait()
        tg = t_v[...]
        cnt = jnp.zeros((L,), jnp.int32)
        for i in range(L):
            ti = jnp.max(jnp.where(iota == i, tg, 0))
            cnt = cnt + jnp.where(tg == ti, 1, 0)
        score = cnt * 2048 + (1023 - tg)
        mx = jnp.max(score)
        pred = 1023 - (mx & 2047)
        o_v[...] = jnp.full((L,), pred, jnp.int32)
        pltpu.sync_copy(o_v, out_hbm)


def kernel(x, train_data, train_targets):
    tt = train_targets.astype(jnp.int32)
    keys, vals = _local_topk(x, train_data)
    out = _final_mode(keys, vals, tt)
    return out[0]


# speedup vs baseline: 1.4212x; 1.0176x over previous
import functools

import jax
import jax.numpy as jnp
from jax import lax
from jax.experimental import pallas as pl
from jax.experimental.pallas import tpu as pltpu
from jax.experimental.pallas import tpu_sc as plsc

N = 100000
D = 128
L = 16
NC = 2
NS = 16
NW = NC * NS
BR = 32
NCH = N // BR
CH_BASE = NCH // NW
CH_REM = NCH % NW
NCC = CH_BASE + 1
NR = 4
SLOTS = NR * ((NCC + NR - 1) // NR)
CH = D // L

_mesh = plsc.VectorSubcoreMesh(core_axis_name="c", subcore_axis_name="s")


def _merge16(tk, tv, bk, bv):
    rk = lax.rev(bk, (0,))
    rv = lax.rev(bv, (0,))
    m = rk < tk
    nk = jnp.where(m, rk, tk)
    nv = jnp.where(m, rv, tv)
    sk, sv = plsc.sort_key_val(nk, nv)
    return sk, sv


@functools.partial(
    pl.kernel,
    mesh=_mesh,
    compiler_params=pltpu.CompilerParams(needs_layout_passes=False),
    out_type=[
        jax.ShapeDtypeStruct((NW, L), jnp.float32),
        jax.ShapeDtypeStruct((NW, L), jnp.int32),
    ],
    scratch_types=[
        pltpu.VMEM((D,), jnp.float32),
        pltpu.VMEM((BR, D), jnp.float32),
        pltpu.VMEM((BR, D), jnp.float32),
        pltpu.VMEM((BR, D), jnp.float32),
        pltpu.VMEM((BR, D), jnp.float32),
        pltpu.VMEM((L,), jnp.float32),
        pltpu.VMEM((L,), jnp.int32),
        pltpu.SemaphoreType.DMA,
        pltpu.SemaphoreType.DMA,
        pltpu.SemaphoreType.DMA,
        pltpu.SemaphoreType.DMA,
    ],
)
def _local_topk(x_hbm, data_hbm, okeys_hbm, ovals_hbm,
                x_v, buf0_v, buf1_v, buf2_v, buf3_v, k_v, v_v,
                sem0, sem1, sem2, sem3):
    wid = lax.axis_index("s") * NC + lax.axis_index("c")
    nch = CH_BASE + jnp.where(wid >= NW - CH_REM, 1, 0)
    start = CH_BASE * wid + jnp.maximum(wid - (NW - CH_REM), 0)

    pltpu.sync_copy(x_hbm, x_v)
    xs = [x_v[pl.ds(c * L, L)] for c in range(CH)]
    iota = lax.broadcasted_iota(jnp.int32, (L,), 0)
    bufs = (buf0_v, buf1_v, buf2_v, buf3_v)
    sems = (sem0, sem1, sem2, sem3)

    def issue(g, b):
        @pl.when(g < NCC)
        def _():
            pltpu.async_copy(
                data_hbm.at[pl.ds((start + g) * BR, BR), :], bufs[b], sems[b])

    for b in range(NR):
        issue(jnp.int32(b), b)

    tk0 = jnp.full((L,), jnp.inf, jnp.float32)
    tv0 = jnp.zeros((L,), jnp.int32)

    def group_dist(buf_v, half):
        sums = []
        for j in range(L):
            sq = []
            for c in range(CH):
                dc = buf_v[half * L + j, pl.ds(c * L, L)] - xs[c]
                sq.append(dc * dc)
            acc = ((sq[0] + sq[1]) + (sq[2] + sq[3])) + (
                (sq[4] + sq[5]) + (sq[6] + sq[7]))
            sums.append(jnp.sum(acc))
        sel = [jnp.where(iota == (2 * j), sums[2 * j], sums[2 * j + 1])
               for j in range(8)]
        sel = [jnp.where((iota >> 1) == (2 * j), sel[2 * j], sel[2 * j + 1])
               for j in range(4)]
        sel = [jnp.where((iota >> 2) == (2 * j), sel[2 * j], sel[2 * j + 1])
               for j in range(2)]
        return jnp.where(iota < 8, sel[0], sel[1])

    def body(gg, carry):
        tk, tv = carry
        for b in range(NR):
            g = gg * NR + b
            buf_v = bufs[b]

            @pl.when(g < NCC)
            def _():
                pltpu.make_async_copy(
                    data_hbm.at[pl.ds(0, BR), :], buf_v, sems[b]).wait()

            row0 = (start + g) * BR
            d0 = group_dist(buf_v, 0)
            d1 = group_dist(buf_v, 1)
            issue(g + NR, b)
            valid = g < nch
            d0 = jnp.where(valid, d0, jnp.inf)
            d1 = jnp.where(valid, d1, jnp.inf)
            bk0, bv0 = plsc.sort_key_val(d0, row0 + iota)
            bk1, bv1 = plsc.sort_key_val(d1, row0 + L + iota)
            bk, bv = _merge16(bk0, bv0, bk1, bv1)
            tk, tv = _merge16(tk, tv, bk, bv)
        return tk, tv

    tk, tv = lax.fori_loop(0, SLOTS // NR, body, (tk0, tv0))
    k_v[...] = tk
    v_v[...] = tv
    pltpu.sync_copy(k_v, okeys_hbm.at[wid])
    pltpu.sync_copy(v_v, ovals_hbm.at[wid])


@functools.partial(
    pl.kernel,
    mesh=_mesh,
    compiler_params=pltpu.CompilerParams(needs_layout_passes=False),
    out_type=jax.ShapeDtypeStruct((L,), jnp.int32),
    scratch_types=[
        pltpu.VMEM((NW, L), jnp.float32),
        pltpu.VMEM((NW, L), jnp.int32),
        pltpu.VMEM((L,), jnp.int32),
        pltpu.VMEM((L,), jnp.int32),
        pltpu.SemaphoreType.DMA,
    ],
)
def _final_mode(keys_hbm, vals_hbm, tgt_hbm, out_hbm, k_v, v_v, t_v, o_v, sem):
    wid = lax.axis_index("s") * NC + lax.axis_index("c")
    iota = lax.broadcasted_iota(jnp.int32, (L,), 0)

    @pl.when(wid == 0)
    def _():
        pltpu.sync_copy(keys_hbm, k_v)
        pltpu.sync_copy(vals_hbm, v_v)
        tk = k_v[0, :]
        tv = v_v[0, :]
        for w in range(1, NW):
            tk, tv = _merge16(tk, tv, k_v[w, :], v_v[w, :])
        pltpu.async_copy(tgt_hbm.at[tv], t_v, sem).wait()
        tg = t_v[...]
        cnt = jnp.zeros((L,), jnp.int32)
        for i in range(L):
            ti = jnp.max(jnp.where(iota == i, tg, 0))
            cnt = cnt + jnp.where(tg == ti, 1, 0)
        score = cnt * 2048 + (1023 - tg)
        mx = jnp.max(score)
        pred = 1023 - (mx & 2047)
        o_v[...] = jnp.full((L,), pred, jnp.int32)
        pltpu.sync_copy(o_v, out_hbm)


def kernel(x, train_data, train_targets):
    tt = train_targets.astype(jnp.int32)
    keys, vals = _local_topk(x, train_data)
    out = _final_mode(keys, vals, tt)
    return out[0]


# speedup vs baseline: 1.4326x; 1.0080x over previous
import functools

import jax
import jax.numpy as jnp
from jax import lax
from jax.experimental import pallas as pl
from jax.experimental.pallas import tpu as pltpu
from jax.experimental.pallas import tpu_sc as plsc

N = 100000
D = 128
L = 16
NC = 2
NS = 16
NW = NC * NS
BR = 32
NCH = N // BR
CH_BASE = NCH // NW
CH_REM = NCH % NW
NCC = CH_BASE + 1
NR = 4
SLOTS = NR * ((NCC + NR - 1) // NR)
CH = D // L

_mesh = plsc.VectorSubcoreMesh(core_axis_name="c", subcore_axis_name="s")


def _merge16(tk, tv, bk, bv):
    rk = lax.rev(bk, (0,))
    rv = lax.rev(bv, (0,))
    m = rk < tk
    nk = jnp.where(m, rk, tk)
    nv = jnp.where(m, rv, tv)
    sk, sv = plsc.sort_key_val(nk, nv)
    return sk, sv


@functools.partial(
    pl.kernel,
    mesh=_mesh,
    compiler_params=pltpu.CompilerParams(needs_layout_passes=False),
    out_type=[
        jax.ShapeDtypeStruct((NW, L), jnp.float32),
        jax.ShapeDtypeStruct((NW, L), jnp.int32),
    ],
    scratch_types=[
        pltpu.VMEM((D,), jnp.float32),
        pltpu.VMEM((BR, D), jnp.float32),
        pltpu.VMEM((BR, D), jnp.float32),
        pltpu.VMEM((BR, D), jnp.float32),
        pltpu.VMEM((BR, D), jnp.float32),
        pltpu.VMEM((L,), jnp.float32),
        pltpu.VMEM((L,), jnp.int32),
        pltpu.SemaphoreType.DMA,
        pltpu.SemaphoreType.DMA,
        pltpu.SemaphoreType.DMA,
        pltpu.SemaphoreType.DMA,
    ],
)
def _local_topk(x_hbm, data_hbm, okeys_hbm, ovals_hbm,
                x_v, buf0_v, buf1_v, buf2_v, buf3_v, k_v, v_v,
                sem0, sem1, sem2, sem3):
    wid = lax.axis_index("s") * NC + lax.axis_index("c")
    nch = CH_BASE + jnp.where(wid >= NW - CH_REM, 1, 0)
    start = CH_BASE * wid + jnp.maximum(wid - (NW - CH_REM), 0)

    pltpu.sync_copy(x_hbm, x_v)
    xs = [x_v[pl.ds(c * L, L)] for c in range(CH)]
    iota = lax.broadcasted_iota(jnp.int32, (L,), 0)
    bufs = (buf0_v, buf1_v, buf2_v, buf3_v)
    sems = (sem0, sem1, sem2, sem3)

    def issue(g, b):
        @pl.when(g < NCC)
        def _():
            pltpu.async_copy(
                data_hbm.at[pl.ds((start + g) * BR, BR), :], bufs[b], sems[b])

    for b in range(NR):
        issue(jnp.int32(b), b)

    tk0 = jnp.full((L,), jnp.inf, jnp.float32)
    tv0 = jnp.zeros((L,), jnp.int32)

    def group_dist(buf_v, half):
        sums = []
        for j in range(L):
            sq = []
            for c in range(CH):
                dc = buf_v[half * L + j, pl.ds(c * L, L)] - xs[c]
                sq.append(dc * dc)
            acc = ((sq[0] + sq[1]) + (sq[2] + sq[3])) + (
                (sq[4] + sq[5]) + (sq[6] + sq[7]))
            sums.append(jnp.sum(acc))
        sel = [jnp.where(iota == (2 * j), sums[2 * j], sums[2 * j + 1])
               for j in range(8)]
        sel = [jnp.where((iota >> 1) == (2 * j), sel[2 * j], sel[2 * j + 1])
               for j in range(4)]
        sel = [jnp.where((iota >> 2) == (2 * j), sel[2 * j], sel[2 * j + 1])
               for j in range(2)]
        return jnp.where(iota < 8, sel[0], sel[1])

    def body(gg, carry):
        tk, tv = carry
        for b in range(NR):
            g = gg * NR + b
            buf_v = bufs[b]

            @pl.when(g < NCC)
            def _():
                pltpu.make_async_copy(
                    data_hbm.at[pl.ds(0, BR), :], buf_v, sems[b]).wait()

            row0 = (start + g) * BR
            d0 = group_dist(buf_v, 0)
            d1 = group_dist(buf_v, 1)
            issue(g + NR, b)
            valid = g < nch
            d0 = jnp.where(valid, d0, jnp.inf)
            d1 = jnp.where(valid, d1, jnp.inf)
            bk0, bv0 = plsc.sort_key_val(d0, row0 + iota)
            bk1, bv1 = plsc.sort_key_val(d1, row0 + L + iota)
            bk, bv = _merge16(bk0, bv0, bk1, bv1)
            tk, tv = _merge16(tk, tv, bk, bv)
        return tk, tv

    tk, tv = lax.fori_loop(0, SLOTS // NR, body, (tk0, tv0))
    k_v[...] = tk
    v_v[...] = tv
    pltpu.sync_copy(k_v, okeys_hbm.at[wid])
    pltpu.sync_copy(v_v, ovals_hbm.at[wid])


@functools.partial(
    pl.kernel,
    mesh=_mesh,
    compiler_params=pltpu.CompilerParams(needs_layout_passes=False),
    out_type=jax.ShapeDtypeStruct((L,), jnp.int32),
    scratch_types=[
        pltpu.VMEM((NW, L), jnp.float32),
        pltpu.VMEM((NW, L), jnp.int32),
        pltpu.VMEM((L,), jnp.int32),
        pltpu.VMEM((L,), jnp.int32),
        pltpu.SemaphoreType.DMA,
    ],
)
def _final_mode(keys_hbm, vals_hbm, tgt_hbm, out_hbm, k_v, v_v, t_v, o_v, sem):
    wid = lax.axis_index("s") * NC + lax.axis_index("c")
    iota = lax.broadcasted_iota(jnp.int32, (L,), 0)

    @pl.when(wid == 0)
    def _():
        pltpu.sync_copy(keys_hbm, k_v)
        pltpu.sync_copy(vals_hbm, v_v)
        lists = [(k_v[w, :], v_v[w, :]) for w in range(NW)]
        while len(lists) > 1:
            lists = [_merge16(lists[i][0], lists[i][1],
                              lists[i + 1][0], lists[i + 1][1])
                     for i in range(0, len(lists), 2)]
        tk, tv = lists[0]
        pltpu.async_copy(tgt_hbm.at[tv], t_v, sem).wait()
        tg = t_v[...]
        cnt = jnp.zeros((L,), jnp.int32)
        for i in range(L):
            ti = jnp.max(jnp.where(iota == i, tg, 0))
            cnt = cnt + jnp.where(tg == ti, 1, 0)
        score = cnt * 2048 + (1023 - tg)
        mx = jnp.max(score)
        pred = 1023 - (mx & 2047)
        o_v[...] = jnp.full((L,), pred, jnp.int32)
        pltpu.sync_copy(o_v, out_hbm)


def kernel(x, train_data, train_targets):
    tt = train_targets.astype(jnp.int32)
    keys, vals = _local_topk(x, train_data)
    out = _final_mode(keys, vals, tt)
    return out[0]
